# baseline (speedup 1.0000x reference)
import jax
import jax.numpy as jnp
from jax.experimental import pallas as pl
from jax.experimental.pallas import tpu as pltpu

_B = 128


def _rot_body(seed_hbm, sw_hbm, c_ref, s_ref, o_ref, seed_v, sw_v, sem):
    i = pl.program_id(0)

    @pl.when(i == 0)
    def _load_seed():
        cp = pltpu.make_async_copy(seed_hbm, seed_v, sem)
        cp.start()
        cp.wait()
        cp2 = pltpu.make_async_copy(sw_hbm, sw_v, sem)
        cp2.start()
        cp2.wait()

    o_ref[...] = seed_v[...] * c_ref[0] + sw_v[...] * s_ref[0]


def kernel(x, pe):
    seq_len = x.shape[-1]
    d = pe.shape[-1]
    nb = seq_len // _B

    seed = pe[:_B]
    sw = jnp.stack([pe[:_B, 1::2], -pe[:_B, 0::2]], axis=-1).reshape(_B, d)
    rows = pe[_B - 1 : (nb - 1) * _B : _B]
    c = jnp.concatenate(
        [jnp.ones((1, d), jnp.float32), jnp.repeat(rows[:, 1::2], 2, axis=1)], axis=0
    ).reshape(nb, 1, d)
    s = jnp.concatenate(
        [jnp.zeros((1, d), jnp.float32), jnp.repeat(rows[:, 0::2], 2, axis=1)], axis=0
    ).reshape(nb, 1, d)

    out = pl.pallas_call(
        _rot_body,
        grid=(nb,),
        in_specs=[
            pl.BlockSpec(memory_space=pl.ANY),
            pl.BlockSpec(memory_space=pl.ANY),
            pl.BlockSpec((1, 1, d), lambda i: (i, 0, 0)),
            pl.BlockSpec((1, 1, d), lambda i: (i, 0, 0)),
        ],
        out_specs=pl.BlockSpec((_B, d), lambda i: (i, 0)),
        out_shape=jax.ShapeDtypeStruct((seq_len, d), jnp.float32),
        scratch_shapes=[
            pltpu.VMEM((_B, d), jnp.float32),
            pltpu.VMEM((_B, d), jnp.float32),
            pltpu.SemaphoreType.DMA,
        ],
    )(seed, sw, c, s)
    return out[None]


# speedup vs baseline: 3.9439x; 3.9439x over previous
import jax
import jax.numpy as jnp
from jax.experimental import pallas as pl
from jax.experimental.pallas import tpu as pltpu

_B = 128


def _rot_body(pe_hbm, o_ref, seed_v, sw_v, cs_v, step_v, sem):
    b = pl.program_id(0)
    d = o_ref.shape[-1]

    @pl.when(b == 0)
    def _init():
        cp = pltpu.make_async_copy(pe_hbm.at[pl.ds(0, _B)], seed_v, sem)
        cp.start()
        cp.wait()
        seed = seed_v[...]
        lane = jax.lax.broadcasted_iota(jnp.int32, (_B, d), 1)
        even = (lane % 2) == 0
        sw_v[...] = jnp.where(
            even, jnp.roll(seed, -1, axis=1), -jnp.roll(seed, 1, axis=1)
        )
        row = seed_v[_B - 1 : _B, :]
        lane1 = jax.lax.broadcasted_iota(jnp.int32, (1, d), 1)
        even1 = (lane1 % 2) == 0
        cstep = jnp.where(even1, jnp.roll(row, -1, axis=1), row)
        sstep = jnp.where(even1, row, jnp.roll(row, 1, axis=1))
        step_v[0:1, :] = cstep
        step_v[1:2, :] = sstep
        cs_v[0:1, :] = jnp.ones((1, d), jnp.float32)
        cs_v[1:2, :] = jnp.zeros((1, d), jnp.float32)

    c = cs_v[0:1, :]
    s = cs_v[1:2, :]
    o_ref[...] = seed_v[...] * c + sw_v[...] * s
    cstep = step_v[0:1, :]
    sstep = step_v[1:2, :]
    cs_v[0:1, :] = c * cstep - s * sstep
    cs_v[1:2, :] = s * cstep + c * sstep


def kernel(x, pe):
    seq_len = x.shape[-1]
    d = pe.shape[-1]
    nb = seq_len // _B

    out = pl.pallas_call(
        _rot_body,
        grid=(nb,),
        in_specs=[pl.BlockSpec(memory_space=pl.ANY)],
        out_specs=pl.BlockSpec((_B, d), lambda i: (i, 0)),
        out_shape=jax.ShapeDtypeStruct((seq_len, d), jnp.float32),
        scratch_shapes=[
            pltpu.VMEM((_B, d), jnp.float32),
            pltpu.VMEM((_B, d), jnp.float32),
            pltpu.VMEM((2, d), jnp.float32),
            pltpu.VMEM((2, d), jnp.float32),
            pltpu.SemaphoreType.DMA,
        ],
    )(pe)
    return out[None]


# speedup vs baseline: 6.3277x; 1.6044x over previous
import jax
import jax.numpy as jnp
from jax.experimental import pallas as pl
from jax.experimental.pallas import tpu as pltpu

_B = 512


def _rot_body(pe_hbm, o_ref, seed_v, sw_v, cs_v, step_v, sem):
    b = pl.program_id(0)
    d = o_ref.shape[-1]

    @pl.when(b == 0)
    def _init():
        cp = pltpu.make_async_copy(pe_hbm.at[pl.ds(0, _B)], seed_v, sem)
        cp.start()
        cp.wait()
        seed = seed_v[...]
        lane = jax.lax.broadcasted_iota(jnp.int32, (_B, d), 1)
        even = (lane % 2) == 0
        sw_v[...] = jnp.where(
            even, jnp.roll(seed, -1, axis=1), -jnp.roll(seed, 1, axis=1)
        )
        row = seed_v[_B - 1 : _B, :]
        lane1 = jax.lax.broadcasted_iota(jnp.int32, (1, d), 1)
        even1 = (lane1 % 2) == 0
        cstep = jnp.where(even1, jnp.roll(row, -1, axis=1), row)
        sstep = jnp.where(even1, row, jnp.roll(row, 1, axis=1))
        step_v[0:1, :] = cstep
        step_v[1:2, :] = sstep
        cs_v[0:1, :] = jnp.ones((1, d), jnp.float32)
        cs_v[1:2, :] = jnp.zeros((1, d), jnp.float32)

    c = cs_v[0:1, :]
    s = cs_v[1:2, :]
    o_ref[...] = seed_v[...] * c + sw_v[...] * s
    cstep = step_v[0:1, :]
    sstep = step_v[1:2, :]
    cs_v[0:1, :] = c * cstep - s * sstep
    cs_v[1:2, :] = s * cstep + c * sstep


def kernel(x, pe):
    seq_len = x.shape[-1]
    d = pe.shape[-1]
    nb = seq_len // _B

    out = pl.pallas_call(
        _rot_body,
        grid=(nb,),
        in_specs=[pl.BlockSpec(memory_space=pl.ANY)],
        out_specs=pl.BlockSpec((_B, d), lambda i: (i, 0)),
        out_shape=jax.ShapeDtypeStruct((seq_len, d), jnp.float32),
        scratch_shapes=[
            pltpu.VMEM((_B, d), jnp.float32),
            pltpu.VMEM((_B, d), jnp.float32),
            pltpu.VMEM((2, d), jnp.float32),
            pltpu.VMEM((2, d), jnp.float32),
            pltpu.SemaphoreType.DMA,
        ],
    )(pe)
    return out[None]
